# Initial kernel scaffold; baseline (speedup 1.0000x reference)
import functools

import jax
import jax.numpy as jnp
from jax import lax
from jax.experimental import pallas as pl
from jax.experimental.pallas import tpu as pltpu
from jax.experimental.pallas import tpu_sc as plsc

N = 10000
D = 128
E = 320000
NG = 64
NCLS = 10

NC = 2
NS = 16
NW = NC * NS
EPW = E // NW
K = 80
NCHUNK = EPW // K
RPS = N // NS
DEGW = 16

BM = 2000


def _make_sc_agg(with_deg):
    mesh = plsc.VectorSubcoreMesh(core_axis_name="c", subcore_axis_name="s")
    out_type = [jax.ShapeDtypeStruct((NC, N, D), jnp.float32)]
    scratch = [
        pltpu.VMEM((K,), jnp.int32),
        pltpu.VMEM((K,), jnp.int32),
        pltpu.VMEM((K, D), jnp.float32),
        pltpu.VMEM_SHARED((N, D), jnp.float32),
    ]
    if with_deg:
        out_type.append(jax.ShapeDtypeStruct((NC, N, DEGW), jnp.float32))
        scratch.append(pltpu.VMEM((K, DEGW), jnp.float32))
        scratch.append(pltpu.VMEM_SHARED((N, DEGW), jnp.float32))

    @functools.partial(
        pl.kernel,
        out_type=tuple(out_type) if with_deg else out_type[0],
        mesh=mesh,
        scratch_types=scratch,
    )
    def sc_agg(*refs):
        it = iter(refs)
        sup_hbm = next(it)
        src_hbm = next(it)
        dst_hbm = next(it)
        znd_hbm = next(it)
        zdeg_hbm = next(it) if with_deg else None
        out_hbm = next(it)
        degout_hbm = next(it) if with_deg else None
        srcv = next(it)
        dstv = next(it)
        rows = next(it)
        acc = next(it)
        if with_deg:
            onesr = next(it)
            dacc = next(it)

        cid = lax.axis_index("c")
        sid = lax.axis_index("s")
        wid = sid * NC + cid
        r0 = sid * RPS

        pltpu.sync_copy(znd_hbm.at[pl.ds(r0, RPS)], acc.at[pl.ds(r0, RPS)])
        if with_deg:
            pltpu.sync_copy(zdeg_hbm.at[pl.ds(r0, RPS)],
                            dacc.at[pl.ds(r0, RPS)])

            @pl.loop(0, K)
            def _(i):
                onesr[i, pl.ds(0, 16)] = jnp.ones((16,), jnp.float32)

        plsc.subcore_barrier()

        base = wid * EPW

        @pl.loop(0, NCHUNK)
        def _(c):
            off = base + c * K
            pltpu.sync_copy(src_hbm.at[pl.ds(off, K)], srcv)
            pltpu.sync_copy(dst_hbm.at[pl.ds(off, K)], dstv)
            pltpu.sync_copy(sup_hbm.at[srcv], rows)
            pltpu.sync_copy(rows, acc.at[dstv], add=True)
            if with_deg:
                pltpu.sync_copy(onesr, dacc.at[dstv], add=True)

        plsc.subcore_barrier()

        pltpu.sync_copy(acc.at[pl.ds(r0, RPS)],
                        out_hbm.at[cid, pl.ds(r0, RPS)])
        if with_deg:
            pltpu.sync_copy(dacc.at[pl.ds(r0, RPS)],
                            degout_hbm.at[cid, pl.ds(r0, RPS)])

    return sc_agg


_sc_agg_deg = _make_sc_agg(True)
_sc_agg = _make_sc_agg(False)


def _mm_body(x_ref, w_ref, o_ref):
    o_ref[...] = jnp.dot(x_ref[...], w_ref[...],
                         preferred_element_type=jnp.float32)


def _matmul(x, w):
    return pl.pallas_call(
        _mm_body,
        grid=(N // BM,),
        in_specs=[pl.BlockSpec((BM, D), lambda i: (i, 0)),
                  pl.BlockSpec((D, D), lambda i: (0, 0))],
        out_specs=pl.BlockSpec((BM, D), lambda i: (i, 0)),
        out_shape=jax.ShapeDtypeStruct((N, D), jnp.float32),
    )(x, w)


def _combine_body(aggp_ref, sup_ref, degp_ref, b1_ref, o_ref):
    deg = degp_ref[0, :, 0:1] + degp_ref[1, :, 0:1] + 1.0
    agg = aggp_ref[0] + aggp_ref[1] + sup_ref[...]
    o_ref[...] = jnp.maximum(agg / deg + b1_ref[...], 0.0)


def _combine_relu(aggp, sup, degp, b1):
    return pl.pallas_call(
        _combine_body,
        grid=(N // BM,),
        in_specs=[pl.BlockSpec((NC, BM, D), lambda i: (0, i, 0)),
                  pl.BlockSpec((BM, D), lambda i: (i, 0)),
                  pl.BlockSpec((NC, BM, DEGW), lambda i: (0, i, 0)),
                  pl.BlockSpec((1, D), lambda i: (0, 0))],
        out_specs=pl.BlockSpec((BM, D), lambda i: (i, 0)),
        out_shape=jax.ShapeDtypeStruct((N, D), jnp.float32),
    )(aggp, sup, degp, b1.reshape(1, D))


def _final_body(aggp_ref, x1_ref, degp_ref, gid_ref, w2_ref, b2_ref,
                mw_ref, mb_ref, o_ref, pooled_acc, cnt_acc):
    i = pl.program_id(0)

    @pl.when(i == 0)
    def _():
        pooled_acc[...] = jnp.zeros_like(pooled_acc)
        cnt_acc[...] = jnp.zeros_like(cnt_acc)

    deg = degp_ref[0, :, 0:1] + degp_ref[1, :, 0:1] + 1.0
    z = (aggp_ref[0] + aggp_ref[1] + x1_ref[...]) / deg
    gids = gid_ref[0, 0, :]
    mask = (lax.broadcasted_iota(jnp.int32, (NG, BM), 0)
            == gids[None, :]).astype(jnp.float32)
    pooled_acc[...] += jnp.dot(mask, z, preferred_element_type=jnp.float32)
    cnt_acc[...] += jnp.sum(mask, axis=1, keepdims=True)

    @pl.when(i == N // BM - 1)
    def _():
        cnt = jnp.maximum(cnt_acc[...], 1.0)
        gr = jnp.dot(pooled_acc[...] / cnt, w2_ref[...],
                     preferred_element_type=jnp.float32) + b2_ref[...]
        logits = jnp.dot(gr, mw_ref[...],
                         preferred_element_type=jnp.float32) + mb_ref[...]
        m = jnp.max(logits, axis=1, keepdims=True)
        lse = jnp.log(jnp.sum(jnp.exp(logits - m), axis=1, keepdims=True)) + m
        o_ref[...] = logits - lse


def _final(aggp, x1, degp, graph_ids, W2, b2, mlp_W, mlp_b):
    gid_r = graph_ids.reshape(N // BM, 1, BM)
    return pl.pallas_call(
        _final_body,
        grid=(N // BM,),
        in_specs=[pl.BlockSpec((NC, BM, D), lambda i: (0, i, 0)),
                  pl.BlockSpec((BM, D), lambda i: (i, 0)),
                  pl.BlockSpec((NC, BM, DEGW), lambda i: (0, i, 0)),
                  pl.BlockSpec((1, 1, BM), lambda i: (i, 0, 0)),
                  pl.BlockSpec((D, D), lambda i: (0, 0)),
                  pl.BlockSpec((1, D), lambda i: (0, 0)),
                  pl.BlockSpec((D, NCLS), lambda i: (0, 0)),
                  pl.BlockSpec((1, NCLS), lambda i: (0, 0))],
        out_specs=pl.BlockSpec((NG, NCLS), lambda i: (0, 0)),
        out_shape=jax.ShapeDtypeStruct((NG, NCLS), jnp.float32),
        scratch_shapes=[pltpu.VMEM((NG, D), jnp.float32),
                        pltpu.VMEM((NG, D), jnp.float32)],
    )(aggp, x1, degp, gid_r, W2, b2.reshape(1, D), mlp_W,
      mlp_b.reshape(1, NCLS))


def kernel(node_feat, W1, b1, W2, b2, mlp_W, mlp_b, edge_index, graph_ids):
    src = edge_index[0]
    dst = edge_index[1]
    znd = jnp.zeros((N, D), jnp.float32)
    zdeg = jnp.zeros((N, DEGW), jnp.float32)

    support1 = _matmul(node_feat, W1)
    aggp1, degp = _sc_agg_deg(support1, src, dst, znd, zdeg)
    x1 = _combine_relu(aggp1, support1, degp, b1)
    aggp2 = _sc_agg(x1, src, dst, znd)
    return _final(aggp2, x1, degp, graph_ids, W2, b2, mlp_W, mlp_b)


# speedup vs baseline: 4.3445x; 4.3445x over previous
import functools

import jax
import jax.numpy as jnp
from jax import lax
from jax.experimental import pallas as pl
from jax.experimental.pallas import tpu as pltpu
from jax.experimental.pallas import tpu_sc as plsc

N = 10000
D = 128
E = 320000
NG = 64
NCLS = 10

NC = 2
NS = 16
NW = NC * NS
EPW = E // NW
K = 80
NCHUNK = EPW // K
NPAD = 10112
RPS = NPAD // NS

BM = 2000

_MESH = plsc.VectorSubcoreMesh(core_axis_name="c", subcore_axis_name="s")


@functools.partial(
    pl.kernel,
    out_type=jax.ShapeDtypeStruct((NC, NPAD, D), jnp.float32),
    mesh=_MESH,
    scratch_types=[
        pltpu.VMEM((K,), jnp.int32),
        pltpu.VMEM((K,), jnp.int32),
        pltpu.VMEM((K, D), jnp.float32),
        pltpu.VMEM_SHARED((NPAD, D), jnp.float32),
        pltpu.SemaphoreType.DMA,
    ],
)
def _sc_agg(sup_hbm, src_hbm, dst_hbm, znd_hbm, out_hbm,
            srcv, dstv, rows, acc, sem):
    cid = lax.axis_index("c")
    sid = lax.axis_index("s")
    wid = sid * NC + cid
    base = wid * EPW
    r0 = sid * RPS

    pltpu.sync_copy(znd_hbm.at[pl.ds(r0, RPS)], acc.at[pl.ds(r0, RPS)])
    plsc.subcore_barrier()

    @pl.loop(0, NCHUNK)
    def _(c):
        off = base + c * K
        pltpu.sync_copy(src_hbm.at[pl.ds(off, K)], srcv)
        pltpu.sync_copy(dst_hbm.at[pl.ds(off, K)], dstv)
        pltpu.async_copy(sup_hbm.at[srcv], rows, sem).wait()
        pltpu.sync_copy(rows, acc.at[dstv], add=True)

    plsc.subcore_barrier()
    pltpu.sync_copy(acc.at[pl.ds(r0, RPS)],
                    out_hbm.at[cid, pl.ds(r0, RPS)])


@functools.partial(
    pl.kernel,
    out_type=jax.ShapeDtypeStruct((NC, NPAD, D), jnp.float32),
    mesh=_MESH,
    scratch_types=[
        pltpu.VMEM((K,), jnp.int32),
        pltpu.VMEM((K, D), jnp.float32),
        pltpu.VMEM_SHARED((NPAD, D), jnp.float32),
    ],
)
def _sc_deg(dst_hbm, ones_hbm, znd_hbm, out_hbm, dstv, onesv, dacc):
    cid = lax.axis_index("c")
    sid = lax.axis_index("s")
    wid = sid * NC + cid
    base = wid * EPW
    r0 = sid * RPS

    pltpu.sync_copy(znd_hbm.at[pl.ds(r0, RPS)], dacc.at[pl.ds(r0, RPS)])
    pltpu.sync_copy(ones_hbm, onesv)
    plsc.subcore_barrier()

    @pl.loop(0, NCHUNK)
    def _(c):
        off = base + c * K
        pltpu.sync_copy(dst_hbm.at[pl.ds(off, K)], dstv)
        pltpu.sync_copy(onesv, dacc.at[dstv], add=True)

    plsc.subcore_barrier()
    pltpu.sync_copy(dacc.at[pl.ds(r0, RPS)],
                    out_hbm.at[cid, pl.ds(r0, RPS)])


def _mm_body(x_ref, w_ref, o_ref):
    o_ref[...] = jnp.dot(x_ref[...], w_ref[...],
                         preferred_element_type=jnp.float32)


def _matmul(x, w):
    return pl.pallas_call(
        _mm_body,
        grid=(N // BM,),
        in_specs=[pl.BlockSpec((BM, D), lambda i: (i, 0)),
                  pl.BlockSpec((D, D), lambda i: (0, 0))],
        out_specs=pl.BlockSpec((BM, D), lambda i: (i, 0)),
        out_shape=jax.ShapeDtypeStruct((N, D), jnp.float32),
    )(x, w)


def _combine_body(aggp_ref, sup_ref, degp_ref, b1_ref, o_ref):
    deg = degp_ref[0, :, 0:1] + degp_ref[1, :, 0:1] + 1.0
    agg = aggp_ref[0] + aggp_ref[1] + sup_ref[...]
    o_ref[...] = jnp.maximum(agg / deg + b1_ref[...], 0.0)


def _combine_relu(aggp, sup, degp, b1):
    return pl.pallas_call(
        _combine_body,
        grid=(N // BM,),
        in_specs=[pl.BlockSpec((NC, BM, D), lambda i: (0, i, 0)),
                  pl.BlockSpec((BM, D), lambda i: (i, 0)),
                  pl.BlockSpec((NC, BM, D), lambda i: (0, i, 0)),
                  pl.BlockSpec((1, D), lambda i: (0, 0))],
        out_specs=pl.BlockSpec((BM, D), lambda i: (i, 0)),
        out_shape=jax.ShapeDtypeStruct((N, D), jnp.float32),
    )(aggp, sup, degp, b1.reshape(1, D))


def _final_body(aggp_ref, x1_ref, degp_ref, gid_ref, w2_ref, b2_ref,
                mw_ref, mb_ref, o_ref, pooled_acc, cnt_acc):
    i = pl.program_id(0)

    @pl.when(i == 0)
    def _():
        pooled_acc[...] = jnp.zeros_like(pooled_acc)
        cnt_acc[...] = jnp.zeros_like(cnt_acc)

    deg = degp_ref[0, :, 0:1] + degp_ref[1, :, 0:1] + 1.0
    z = (aggp_ref[0] + aggp_ref[1] + x1_ref[...]) / deg
    gids = gid_ref[0, 0, :]
    mask = (lax.broadcasted_iota(jnp.int32, (NG, BM), 0)
            == gids[None, :]).astype(jnp.float32)
    pooled_acc[...] += jnp.dot(mask, z, preferred_element_type=jnp.float32)
    cnt_acc[...] += jnp.sum(mask, axis=1, keepdims=True)

    @pl.when(i == N // BM - 1)
    def _():
        cnt = jnp.maximum(cnt_acc[...], 1.0)
        gr = jnp.dot(pooled_acc[...] / cnt, w2_ref[...],
                     preferred_element_type=jnp.float32) + b2_ref[...]
        logits = jnp.dot(gr, mw_ref[...],
                         preferred_element_type=jnp.float32) + mb_ref[...]
        m = jnp.max(logits, axis=1, keepdims=True)
        lse = jnp.log(jnp.sum(jnp.exp(logits - m), axis=1, keepdims=True)) + m
        o_ref[...] = logits - lse


def _final(aggp, x1, degp, graph_ids, W2, b2, mlp_W, mlp_b):
    gid_r = graph_ids.reshape(N // BM, 1, BM)
    return pl.pallas_call(
        _final_body,
        grid=(N // BM,),
        in_specs=[pl.BlockSpec((NC, BM, D), lambda i: (0, i, 0)),
                  pl.BlockSpec((BM, D), lambda i: (i, 0)),
                  pl.BlockSpec((NC, BM, D), lambda i: (0, i, 0)),
                  pl.BlockSpec((1, 1, BM), lambda i: (i, 0, 0)),
                  pl.BlockSpec((D, D), lambda i: (0, 0)),
                  pl.BlockSpec((1, D), lambda i: (0, 0)),
                  pl.BlockSpec((D, NCLS), lambda i: (0, 0)),
                  pl.BlockSpec((1, NCLS), lambda i: (0, 0))],
        out_specs=pl.BlockSpec((NG, NCLS), lambda i: (0, 0)),
        out_shape=jax.ShapeDtypeStruct((NG, NCLS), jnp.float32),
        scratch_shapes=[pltpu.VMEM((NG, D), jnp.float32),
                        pltpu.VMEM((NG, D), jnp.float32)],
    )(aggp, x1, degp, gid_r, W2, b2.reshape(1, D), mlp_W,
      mlp_b.reshape(1, NCLS))


def kernel(node_feat, W1, b1, W2, b2, mlp_W, mlp_b, edge_index, graph_ids):
    src = edge_index[0]
    dst = edge_index[1]
    znd = jnp.zeros((NPAD, D), jnp.float32)
    ones = jnp.ones((K, D), jnp.float32)

    support1 = _matmul(node_feat, W1)
    degp = _sc_deg(dst, ones, znd)
    aggp1 = _sc_agg(support1, src, dst, znd)
    x1 = _combine_relu(aggp1, support1, degp, b1)
    aggp2 = _sc_agg(x1, src, dst, znd)
    return _final(aggp2, x1, degp, graph_ids, W2, b2, mlp_W, mlp_b)


# speedup vs baseline: 7.2500x; 1.6688x over previous
import functools

import jax
import jax.numpy as jnp
from jax import lax
from jax.experimental import pallas as pl
from jax.experimental.pallas import tpu as pltpu
from jax.experimental.pallas import tpu_sc as plsc

N = 10000
D = 128
E = 320000
NG = 64
NCLS = 10

NC = 2
NS = 16
NW = NC * NS
EPW = E // NW
K = 80
NCHUNK = EPW // K
NB = 4
NBATCH = NCHUNK // NB
NPAD = 10112
RPS = NPAD // NS

BM = 2000

_MESH = plsc.VectorSubcoreMesh(core_axis_name="c", subcore_axis_name="s")


@functools.partial(
    pl.kernel,
    out_type=jax.ShapeDtypeStruct((NC, NPAD, D), jnp.float32),
    mesh=_MESH,
    scratch_types=[
        pltpu.VMEM((NB, K), jnp.int32),
        pltpu.VMEM((NB, K), jnp.int32),
        pltpu.VMEM((NB, K, D), jnp.float32),
        pltpu.VMEM_SHARED((NPAD, D), jnp.float32),
        pltpu.SemaphoreType.DMA,
        pltpu.SemaphoreType.DMA,
    ],
)
def _sc_agg(sup_hbm, src_hbm, dst_hbm, znd_hbm, out_hbm,
            srcb, dstb, rows, acc, semi, semg):
    cid = lax.axis_index("c")
    sid = lax.axis_index("s")
    wid = sid * NC + cid
    base = wid * EPW
    r0 = sid * RPS

    pltpu.sync_copy(znd_hbm.at[pl.ds(r0, RPS)], acc.at[pl.ds(r0, RPS)])
    plsc.subcore_barrier()

    @pl.loop(0, NBATCH)
    def _(j):
        c0 = j * NB
        hs = []
        for b in range(NB):
            off = base + (c0 + b) * K
            hs.append(pltpu.async_copy(src_hbm.at[pl.ds(off, K)],
                                       srcb.at[b], semi))
            hs.append(pltpu.async_copy(dst_hbm.at[pl.ds(off, K)],
                                       dstb.at[b], semi))
        for h in hs:
            h.wait()
        gs = [pltpu.async_copy(sup_hbm.at[srcb.at[b]], rows.at[b], semg)
              for b in range(NB)]
        for g in gs:
            g.wait()
        for b in range(NB):
            pltpu.sync_copy(rows.at[b], acc.at[dstb.at[b]], add=True)

    for t in range(NB * NBATCH, NCHUNK):
        off = base + t * K
        pltpu.sync_copy(src_hbm.at[pl.ds(off, K)], srcb.at[0])
        pltpu.sync_copy(dst_hbm.at[pl.ds(off, K)], dstb.at[0])
        pltpu.async_copy(sup_hbm.at[srcb.at[0]], rows.at[0], semg).wait()
        pltpu.sync_copy(rows.at[0], acc.at[dstb.at[0]], add=True)

    plsc.subcore_barrier()
    pltpu.sync_copy(acc.at[pl.ds(r0, RPS)],
                    out_hbm.at[cid, pl.ds(r0, RPS)])


@functools.partial(
    pl.kernel,
    out_type=jax.ShapeDtypeStruct((NC, NPAD, D), jnp.float32),
    mesh=_MESH,
    scratch_types=[
        pltpu.VMEM((NB, K), jnp.int32),
        pltpu.VMEM((K, D), jnp.float32),
        pltpu.VMEM_SHARED((NPAD, D), jnp.float32),
        pltpu.SemaphoreType.DMA,
    ],
)
def _sc_deg(dst_hbm, ones_hbm, znd_hbm, out_hbm, dstb, onesv, dacc, semi):
    cid = lax.axis_index("c")
    sid = lax.axis_index("s")
    wid = sid * NC + cid
    base = wid * EPW
    r0 = sid * RPS

    pltpu.sync_copy(znd_hbm.at[pl.ds(r0, RPS)], dacc.at[pl.ds(r0, RPS)])
    pltpu.sync_copy(ones_hbm, onesv)
    plsc.subcore_barrier()

    @pl.loop(0, NBATCH)
    def _(j):
        c0 = j * NB
        hs = [pltpu.async_copy(dst_hbm.at[pl.ds(base + (c0 + b) * K, K)],
                               dstb.at[b], semi) for b in range(NB)]
        for h in hs:
            h.wait()
        for b in range(NB):
            pltpu.sync_copy(onesv, dacc.at[dstb.at[b]], add=True)

    for t in range(NB * NBATCH, NCHUNK):
        pltpu.sync_copy(dst_hbm.at[pl.ds(base + t * K, K)], dstb.at[0])
        pltpu.sync_copy(onesv, dacc.at[dstb.at[0]], add=True)

    plsc.subcore_barrier()
    pltpu.sync_copy(dacc.at[pl.ds(r0, RPS)],
                    out_hbm.at[cid, pl.ds(r0, RPS)])


def _mm_body(x_ref, w_ref, o_ref):
    o_ref[...] = jnp.dot(x_ref[...], w_ref[...],
                         preferred_element_type=jnp.float32)


def _matmul(x, w):
    return pl.pallas_call(
        _mm_body,
        grid=(N // BM,),
        in_specs=[pl.BlockSpec((BM, D), lambda i: (i, 0)),
                  pl.BlockSpec((D, D), lambda i: (0, 0))],
        out_specs=pl.BlockSpec((BM, D), lambda i: (i, 0)),
        out_shape=jax.ShapeDtypeStruct((N, D), jnp.float32),
    )(x, w)


def _combine_body(aggp_ref, sup_ref, degp_ref, b1_ref, o_ref):
    deg = degp_ref[0, :, 0:1] + degp_ref[1, :, 0:1] + 1.0
    agg = aggp_ref[0] + aggp_ref[1] + sup_ref[...]
    o_ref[...] = jnp.maximum(agg / deg + b1_ref[...], 0.0)


def _combine_relu(aggp, sup, degp, b1):
    return pl.pallas_call(
        _combine_body,
        grid=(N // BM,),
        in_specs=[pl.BlockSpec((NC, BM, D), lambda i: (0, i, 0)),
                  pl.BlockSpec((BM, D), lambda i: (i, 0)),
                  pl.BlockSpec((NC, BM, D), lambda i: (0, i, 0)),
                  pl.BlockSpec((1, D), lambda i: (0, 0))],
        out_specs=pl.BlockSpec((BM, D), lambda i: (i, 0)),
        out_shape=jax.ShapeDtypeStruct((N, D), jnp.float32),
    )(aggp, sup, degp, b1.reshape(1, D))


def _final_body(aggp_ref, x1_ref, degp_ref, gid_ref, w2_ref, b2_ref,
                mw_ref, mb_ref, o_ref, pooled_acc, cnt_acc):
    i = pl.program_id(0)

    @pl.when(i == 0)
    def _():
        pooled_acc[...] = jnp.zeros_like(pooled_acc)
        cnt_acc[...] = jnp.zeros_like(cnt_acc)

    deg = degp_ref[0, :, 0:1] + degp_ref[1, :, 0:1] + 1.0
    z = (aggp_ref[0] + aggp_ref[1] + x1_ref[...]) / deg
    gids = gid_ref[0, 0, :]
    mask = (lax.broadcasted_iota(jnp.int32, (NG, BM), 0)
            == gids[None, :]).astype(jnp.float32)
    pooled_acc[...] += jnp.dot(mask, z, preferred_element_type=jnp.float32)
    cnt_acc[...] += jnp.sum(mask, axis=1, keepdims=True)

    @pl.when(i == N // BM - 1)
    def _():
        cnt = jnp.maximum(cnt_acc[...], 1.0)
        gr = jnp.dot(pooled_acc[...] / cnt, w2_ref[...],
                     preferred_element_type=jnp.float32) + b2_ref[...]
        logits = jnp.dot(gr, mw_ref[...],
                         preferred_element_type=jnp.float32) + mb_ref[...]
        m = jnp.max(logits, axis=1, keepdims=True)
        lse = jnp.log(jnp.sum(jnp.exp(logits - m), axis=1, keepdims=True)) + m
        o_ref[...] = logits - lse


def _final(aggp, x1, degp, graph_ids, W2, b2, mlp_W, mlp_b):
    gid_r = graph_ids.reshape(N // BM, 1, BM)
    return pl.pallas_call(
        _final_body,
        grid=(N // BM,),
        in_specs=[pl.BlockSpec((NC, BM, D), lambda i: (0, i, 0)),
                  pl.BlockSpec((BM, D), lambda i: (i, 0)),
                  pl.BlockSpec((NC, BM, D), lambda i: (0, i, 0)),
                  pl.BlockSpec((1, 1, BM), lambda i: (i, 0, 0)),
                  pl.BlockSpec((D, D), lambda i: (0, 0)),
                  pl.BlockSpec((1, D), lambda i: (0, 0)),
                  pl.BlockSpec((D, NCLS), lambda i: (0, 0)),
                  pl.BlockSpec((1, NCLS), lambda i: (0, 0))],
        out_specs=pl.BlockSpec((NG, NCLS), lambda i: (0, 0)),
        out_shape=jax.ShapeDtypeStruct((NG, NCLS), jnp.float32),
        scratch_shapes=[pltpu.VMEM((NG, D), jnp.float32),
                        pltpu.VMEM((NG, D), jnp.float32)],
    )(aggp, x1, degp, gid_r, W2, b2.reshape(1, D), mlp_W,
      mlp_b.reshape(1, NCLS))


def kernel(node_feat, W1, b1, W2, b2, mlp_W, mlp_b, edge_index, graph_ids):
    src = edge_index[0]
    dst = edge_index[1]
    znd = jnp.zeros((NPAD, D), jnp.float32)
    ones = jnp.ones((K, D), jnp.float32)

    support1 = _matmul(node_feat, W1)
    degp = _sc_deg(dst, ones, znd)
    aggp1 = _sc_agg(support1, src, dst, znd)
    x1 = _combine_relu(aggp1, support1, degp, b1)
    aggp2 = _sc_agg(x1, src, dst, znd)
    return _final(aggp2, x1, degp, graph_ids, W2, b2, mlp_W, mlp_b)


# speedup vs baseline: 8.7794x; 1.2109x over previous
import functools

import jax
import jax.numpy as jnp
from jax import lax
from jax.experimental import pallas as pl
from jax.experimental.pallas import tpu as pltpu
from jax.experimental.pallas import tpu_sc as plsc

N = 10000
D = 128
E = 320000
NG = 64
NCLS = 10

NC = 2
NS = 16
NW = NC * NS
EPW = E // NW
K = 80
NCHUNK = EPW // K
NB = 4
NBATCH = NCHUNK // NB
NPAIR = (NCHUNK - 2) // 4
NPAD = 10112
RPS = NPAD // NS

BM = 2000

_MESH = plsc.VectorSubcoreMesh(core_axis_name="c", subcore_axis_name="s")


@functools.partial(
    pl.kernel,
    out_type=jax.ShapeDtypeStruct((NC, NPAD, D), jnp.float32),
    mesh=_MESH,
    scratch_types=[
        pltpu.VMEM((NB, K), jnp.int32),
        pltpu.VMEM((NB, K), jnp.int32),
        pltpu.VMEM((NB, K, D), jnp.float32),
        pltpu.VMEM_SHARED((NPAD, D), jnp.float32),
        pltpu.SemaphoreType.DMA,
        pltpu.SemaphoreType.DMA,
    ],
)
def _sc_agg(sup_hbm, src_hbm, dst_hbm, znd_hbm, out_hbm,
            srcb, dstb, rows, acc, semi, semg):
    cid = lax.axis_index("c")
    sid = lax.axis_index("s")
    wid = sid * NC + cid
    base = wid * EPW
    r0 = sid * RPS

    pltpu.sync_copy(znd_hbm.at[pl.ds(r0, RPS)], acc.at[pl.ds(r0, RPS)])
    plsc.subcore_barrier()

    def _fire_idx(c, s):
        off = base + c * K
        pltpu.async_copy(src_hbm.at[pl.ds(off, K)], srcb.at[s], semi)
        pltpu.async_copy(dst_hbm.at[pl.ds(off, K)], dstb.at[s], semi)

    def _drain_idx(c, s):
        off = base + c * K
        pltpu.make_async_copy(src_hbm.at[pl.ds(off, K)], srcb.at[s],
                              semi).wait()
        pltpu.make_async_copy(dst_hbm.at[pl.ds(off, K)], dstb.at[s],
                              semi).wait()

    def _fire_gather(s):
        pltpu.async_copy(sup_hbm.at[srcb.at[s]], rows.at[s], semg)

    def _drain_gather(s):
        pltpu.make_async_copy(sup_hbm.at[srcb.at[s]], rows.at[s],
                              semg).wait()

    def _scatter(s):
        pltpu.sync_copy(rows.at[s], acc.at[dstb.at[s]], add=True)

    _fire_idx(0, 0)
    _fire_idx(1, 1)
    _drain_idx(0, 0)
    _drain_idx(1, 1)
    _fire_gather(0)
    _fire_gather(1)

    @pl.loop(0, NPAIR)
    def _(j):
        q = j * 4
        _fire_idx(q + 2, 2)
        _fire_idx(q + 3, 3)
        _drain_gather(0)
        _drain_gather(1)
        _drain_idx(q + 2, 2)
        _drain_idx(q + 3, 3)
        _fire_gather(2)
        _fire_gather(3)
        _scatter(0)
        _scatter(1)
        _fire_idx(q + 4, 0)
        _fire_idx(q + 5, 1)
        _drain_gather(2)
        _drain_gather(3)
        _drain_idx(q + 4, 0)
        _drain_idx(q + 5, 1)
        _fire_gather(0)
        _fire_gather(1)
        _scatter(2)
        _scatter(3)

    _drain_gather(0)
    _drain_gather(1)
    _scatter(0)
    _scatter(1)
    for t in range(NPAIR * 4 + 2, NCHUNK):
        _fire_idx(t, 0)
        _drain_idx(t, 0)
        _fire_gather(0)
        _drain_gather(0)
        _scatter(0)

    plsc.subcore_barrier()
    pltpu.sync_copy(acc.at[pl.ds(r0, RPS)],
                    out_hbm.at[cid, pl.ds(r0, RPS)])


@functools.partial(
    pl.kernel,
    out_type=jax.ShapeDtypeStruct((NC, NPAD, D), jnp.float32),
    mesh=_MESH,
    scratch_types=[
        pltpu.VMEM((NB, K), jnp.int32),
        pltpu.VMEM((K, D), jnp.float32),
        pltpu.VMEM_SHARED((NPAD, D), jnp.float32),
        pltpu.SemaphoreType.DMA,
    ],
)
def _sc_deg(dst_hbm, ones_hbm, znd_hbm, out_hbm, dstb, onesv, dacc, semi):
    cid = lax.axis_index("c")
    sid = lax.axis_index("s")
    wid = sid * NC + cid
    base = wid * EPW
    r0 = sid * RPS

    pltpu.sync_copy(znd_hbm.at[pl.ds(r0, RPS)], dacc.at[pl.ds(r0, RPS)])
    pltpu.sync_copy(ones_hbm, onesv)
    plsc.subcore_barrier()

    @pl.loop(0, NBATCH)
    def _(j):
        c0 = j * NB
        hs = [pltpu.async_copy(dst_hbm.at[pl.ds(base + (c0 + b) * K, K)],
                               dstb.at[b], semi) for b in range(NB)]
        for h in hs:
            h.wait()
        for b in range(NB):
            pltpu.sync_copy(onesv, dacc.at[dstb.at[b]], add=True)

    for t in range(NB * NBATCH, NCHUNK):
        pltpu.sync_copy(dst_hbm.at[pl.ds(base + t * K, K)], dstb.at[0])
        pltpu.sync_copy(onesv, dacc.at[dstb.at[0]], add=True)

    plsc.subcore_barrier()
    pltpu.sync_copy(dacc.at[pl.ds(r0, RPS)],
                    out_hbm.at[cid, pl.ds(r0, RPS)])


def _mm_body(x_ref, w_ref, o_ref):
    o_ref[...] = jnp.dot(x_ref[...], w_ref[...],
                         preferred_element_type=jnp.float32)


def _matmul(x, w):
    return pl.pallas_call(
        _mm_body,
        grid=(N // BM,),
        in_specs=[pl.BlockSpec((BM, D), lambda i: (i, 0)),
                  pl.BlockSpec((D, D), lambda i: (0, 0))],
        out_specs=pl.BlockSpec((BM, D), lambda i: (i, 0)),
        out_shape=jax.ShapeDtypeStruct((N, D), jnp.float32),
    )(x, w)


def _combine_body(aggp_ref, sup_ref, degp_ref, b1_ref, o_ref):
    deg = degp_ref[0, :, 0:1] + degp_ref[1, :, 0:1] + 1.0
    agg = aggp_ref[0] + aggp_ref[1] + sup_ref[...]
    o_ref[...] = jnp.maximum(agg / deg + b1_ref[...], 0.0)


def _combine_relu(aggp, sup, degp, b1):
    return pl.pallas_call(
        _combine_body,
        grid=(N // BM,),
        in_specs=[pl.BlockSpec((NC, BM, D), lambda i: (0, i, 0)),
                  pl.BlockSpec((BM, D), lambda i: (i, 0)),
                  pl.BlockSpec((NC, BM, D), lambda i: (0, i, 0)),
                  pl.BlockSpec((1, D), lambda i: (0, 0))],
        out_specs=pl.BlockSpec((BM, D), lambda i: (i, 0)),
        out_shape=jax.ShapeDtypeStruct((N, D), jnp.float32),
    )(aggp, sup, degp, b1.reshape(1, D))


def _final_body(aggp_ref, x1_ref, degp_ref, gid_ref, w2_ref, b2_ref,
                mw_ref, mb_ref, o_ref, pooled_acc, cnt_acc):
    i = pl.program_id(0)

    @pl.when(i == 0)
    def _():
        pooled_acc[...] = jnp.zeros_like(pooled_acc)
        cnt_acc[...] = jnp.zeros_like(cnt_acc)

    deg = degp_ref[0, :, 0:1] + degp_ref[1, :, 0:1] + 1.0
    z = (aggp_ref[0] + aggp_ref[1] + x1_ref[...]) / deg
    gids = gid_ref[0, 0, :]
    mask = (lax.broadcasted_iota(jnp.int32, (NG, BM), 0)
            == gids[None, :]).astype(jnp.float32)
    pooled_acc[...] += jnp.dot(mask, z, preferred_element_type=jnp.float32)
    cnt_acc[...] += jnp.sum(mask, axis=1, keepdims=True)

    @pl.when(i == N // BM - 1)
    def _():
        cnt = jnp.maximum(cnt_acc[...], 1.0)
        gr = jnp.dot(pooled_acc[...] / cnt, w2_ref[...],
                     preferred_element_type=jnp.float32) + b2_ref[...]
        logits = jnp.dot(gr, mw_ref[...],
                         preferred_element_type=jnp.float32) + mb_ref[...]
        m = jnp.max(logits, axis=1, keepdims=True)
        lse = jnp.log(jnp.sum(jnp.exp(logits - m), axis=1, keepdims=True)) + m
        o_ref[...] = logits - lse


def _final(aggp, x1, degp, graph_ids, W2, b2, mlp_W, mlp_b):
    gid_r = graph_ids.reshape(N // BM, 1, BM)
    return pl.pallas_call(
        _final_body,
        grid=(N // BM,),
        in_specs=[pl.BlockSpec((NC, BM, D), lambda i: (0, i, 0)),
                  pl.BlockSpec((BM, D), lambda i: (i, 0)),
                  pl.BlockSpec((NC, BM, D), lambda i: (0, i, 0)),
                  pl.BlockSpec((1, 1, BM), lambda i: (i, 0, 0)),
                  pl.BlockSpec((D, D), lambda i: (0, 0)),
                  pl.BlockSpec((1, D), lambda i: (0, 0)),
                  pl.BlockSpec((D, NCLS), lambda i: (0, 0)),
                  pl.BlockSpec((1, NCLS), lambda i: (0, 0))],
        out_specs=pl.BlockSpec((NG, NCLS), lambda i: (0, 0)),
        out_shape=jax.ShapeDtypeStruct((NG, NCLS), jnp.float32),
        scratch_shapes=[pltpu.VMEM((NG, D), jnp.float32),
                        pltpu.VMEM((NG, D), jnp.float32)],
    )(aggp, x1, degp, gid_r, W2, b2.reshape(1, D), mlp_W,
      mlp_b.reshape(1, NCLS))


def kernel(node_feat, W1, b1, W2, b2, mlp_W, mlp_b, edge_index, graph_ids):
    src = edge_index[0]
    dst = edge_index[1]
    znd = jnp.zeros((NPAD, D), jnp.float32)
    ones = jnp.ones((K, D), jnp.float32)

    support1 = _matmul(node_feat, W1)
    degp = _sc_deg(dst, ones, znd)
    aggp1 = _sc_agg(support1, src, dst, znd)
    x1 = _combine_relu(aggp1, support1, degp, b1)
    aggp2 = _sc_agg(x1, src, dst, znd)
    return _final(aggp2, x1, degp, graph_ids, W2, b2, mlp_W, mlp_b)


# speedup vs baseline: 8.9301x; 1.0172x over previous
import functools

import jax
import jax.numpy as jnp
from jax import lax
from jax.experimental import pallas as pl
from jax.experimental.pallas import tpu as pltpu
from jax.experimental.pallas import tpu_sc as plsc

N = 10000
D = 128
E = 320000
NG = 64
NCLS = 10

NC = 2
NS = 16
NW = NC * NS
EPW = E // NW
K = 80
NCHUNK = EPW // K
NB = 4
NBATCH = NCHUNK // NB
NPAIR = (NCHUNK - 2) // 4
NPAD = 10112
RPS = NPAD // NS

DW = 128
BM = 2000

_MESH = plsc.VectorSubcoreMesh(core_axis_name="c", subcore_axis_name="s")


@functools.partial(
    pl.kernel,
    out_type=jax.ShapeDtypeStruct((NC, NPAD, D), jnp.float32),
    mesh=_MESH,
    scratch_types=[
        pltpu.VMEM((NB, K), jnp.int32),
        pltpu.VMEM((NB, K), jnp.int32),
        pltpu.VMEM((NB, K, D), jnp.float32),
        pltpu.VMEM_SHARED((NPAD, D), jnp.float32),
        pltpu.SemaphoreType.DMA,
        pltpu.SemaphoreType.DMA,
        pltpu.SemaphoreType.DMA,
    ],
)
def _sc_agg(sup_hbm, src_hbm, dst_hbm, znd_hbm, out_hbm,
            srcb, dstb, rows, acc, semi, semg, sems):
    cid = lax.axis_index("c")
    sid = lax.axis_index("s")
    wid = sid * NC + cid
    base = wid * EPW
    r0 = sid * RPS

    pltpu.sync_copy(znd_hbm.at[pl.ds(r0, RPS)], acc.at[pl.ds(r0, RPS)])
    plsc.subcore_barrier()

    def _fire_idx(c, s):
        off = base + c * K
        pltpu.async_copy(src_hbm.at[pl.ds(off, K)], srcb.at[s], semi)
        pltpu.async_copy(dst_hbm.at[pl.ds(off, K)], dstb.at[s], semi)

    def _drain_idx(c, s):
        off = base + c * K
        pltpu.make_async_copy(src_hbm.at[pl.ds(off, K)], srcb.at[s],
                              semi).wait()
        pltpu.make_async_copy(dst_hbm.at[pl.ds(off, K)], dstb.at[s],
                              semi).wait()

    def _fire_gather(s):
        pltpu.async_copy(sup_hbm.at[srcb.at[s]], rows.at[s], semg)

    def _drain_gather(s):
        pltpu.make_async_copy(sup_hbm.at[srcb.at[s]], rows.at[s],
                              semg).wait()

    def _scatter(s):
        pltpu.sync_copy(rows.at[s], acc.at[dstb.at[s]], add=True)

    def _fire_scatter(s):
        pltpu.async_copy(rows.at[s], acc.at[dstb.at[s]], sems, add=True)

    def _drain_scatter(s):
        pltpu.make_async_copy(rows.at[s], acc.at[dstb.at[s]], sems).wait()

    _fire_idx(0, 0)
    _fire_idx(1, 1)
    _drain_idx(0, 0)
    _drain_idx(1, 1)
    _fire_gather(0)
    _fire_gather(1)

    @pl.loop(0, NPAIR)
    def _(j):
        q = j * 4
        _fire_idx(q + 2, 2)
        _fire_idx(q + 3, 3)
        _drain_gather(0)
        _fire_scatter(0)
        _drain_gather(1)
        _fire_scatter(1)
        _drain_idx(q + 2, 2)
        _drain_idx(q + 3, 3)
        _fire_gather(2)
        _fire_gather(3)
        _drain_scatter(0)
        _drain_scatter(1)
        _fire_idx(q + 4, 0)
        _fire_idx(q + 5, 1)
        _drain_gather(2)
        _fire_scatter(2)
        _drain_gather(3)
        _fire_scatter(3)
        _drain_idx(q + 4, 0)
        _drain_idx(q + 5, 1)
        _fire_gather(0)
        _fire_gather(1)
        _drain_scatter(2)
        _drain_scatter(3)

    _drain_gather(0)
    _drain_gather(1)
    _scatter(0)
    _scatter(1)
    for t in range(NPAIR * 4 + 2, NCHUNK):
        _fire_idx(t, 0)
        _drain_idx(t, 0)
        _fire_gather(0)
        _drain_gather(0)
        _scatter(0)

    plsc.subcore_barrier()
    pltpu.sync_copy(acc.at[pl.ds(r0, RPS)],
                    out_hbm.at[cid, pl.ds(r0, RPS)])


@functools.partial(
    pl.kernel,
    out_type=jax.ShapeDtypeStruct((NC, NPAD, DW), jnp.float32),
    mesh=_MESH,
    scratch_types=[
        pltpu.VMEM((NB, K), jnp.int32),
        pltpu.VMEM((K, DW), jnp.float32),
        pltpu.VMEM_SHARED((NPAD, DW), jnp.float32),
        pltpu.SemaphoreType.DMA,
        pltpu.SemaphoreType.DMA,
    ],
)
def _sc_deg(dst_hbm, ones_hbm, znd_hbm, out_hbm, dstb, onesv, dacc, semi,
            sems):
    cid = lax.axis_index("c")
    sid = lax.axis_index("s")
    wid = sid * NC + cid
    base = wid * EPW
    r0 = sid * RPS

    pltpu.sync_copy(znd_hbm.at[pl.ds(r0, RPS)], dacc.at[pl.ds(r0, RPS)])
    pltpu.sync_copy(ones_hbm, onesv)
    plsc.subcore_barrier()

    @pl.loop(0, NBATCH)
    def _(j):
        c0 = j * NB
        hs = [pltpu.async_copy(dst_hbm.at[pl.ds(base + (c0 + b) * K, K)],
                               dstb.at[b], semi) for b in range(NB)]
        for h in hs:
            h.wait()
        ss = [pltpu.async_copy(onesv, dacc.at[dstb.at[b]], sems, add=True)
              for b in range(NB)]
        for s in ss:
            s.wait()

    for t in range(NB * NBATCH, NCHUNK):
        pltpu.sync_copy(dst_hbm.at[pl.ds(base + t * K, K)], dstb.at[0])
        pltpu.sync_copy(onesv, dacc.at[dstb.at[0]], add=True)

    plsc.subcore_barrier()
    pltpu.sync_copy(dacc.at[pl.ds(r0, RPS)],
                    out_hbm.at[cid, pl.ds(r0, RPS)])


def _mm_body(x_ref, w_ref, o_ref):
    o_ref[...] = jnp.dot(x_ref[...], w_ref[...],
                         preferred_element_type=jnp.float32)


def _matmul(x, w):
    return pl.pallas_call(
        _mm_body,
        grid=(N // BM,),
        in_specs=[pl.BlockSpec((BM, D), lambda i: (i, 0)),
                  pl.BlockSpec((D, D), lambda i: (0, 0))],
        out_specs=pl.BlockSpec((BM, D), lambda i: (i, 0)),
        out_shape=jax.ShapeDtypeStruct((N, D), jnp.float32),
    )(x, w)


def _combine_body(aggp_ref, sup_ref, degp_ref, b1_ref, o_ref):
    deg = degp_ref[0, :, 0:1] + degp_ref[1, :, 0:1] + 1.0
    agg = aggp_ref[0] + aggp_ref[1] + sup_ref[...]
    o_ref[...] = jnp.maximum(agg / deg + b1_ref[...], 0.0)


def _combine_relu(aggp, sup, degp, b1):
    return pl.pallas_call(
        _combine_body,
        grid=(N // BM,),
        in_specs=[pl.BlockSpec((NC, BM, D), lambda i: (0, i, 0)),
                  pl.BlockSpec((BM, D), lambda i: (i, 0)),
                  pl.BlockSpec((NC, BM, DW), lambda i: (0, i, 0)),
                  pl.BlockSpec((1, D), lambda i: (0, 0))],
        out_specs=pl.BlockSpec((BM, D), lambda i: (i, 0)),
        out_shape=jax.ShapeDtypeStruct((N, D), jnp.float32),
    )(aggp, sup, degp, b1.reshape(1, D))


def _final_body(aggp_ref, x1_ref, degp_ref, gid_ref, w2_ref, b2_ref,
                mw_ref, mb_ref, o_ref, pooled_acc, cnt_acc):
    i = pl.program_id(0)

    @pl.when(i == 0)
    def _():
        pooled_acc[...] = jnp.zeros_like(pooled_acc)
        cnt_acc[...] = jnp.zeros_like(cnt_acc)

    deg = degp_ref[0, :, 0:1] + degp_ref[1, :, 0:1] + 1.0
    z = (aggp_ref[0] + aggp_ref[1] + x1_ref[...]) / deg
    gids = gid_ref[0, 0, :]
    mask = (lax.broadcasted_iota(jnp.int32, (NG, BM), 0)
            == gids[None, :]).astype(jnp.float32)
    pooled_acc[...] += jnp.dot(mask, z, preferred_element_type=jnp.float32)
    cnt_acc[...] += jnp.sum(mask, axis=1, keepdims=True)

    @pl.when(i == N // BM - 1)
    def _():
        cnt = jnp.maximum(cnt_acc[...], 1.0)
        gr = jnp.dot(pooled_acc[...] / cnt, w2_ref[...],
                     preferred_element_type=jnp.float32) + b2_ref[...]
        logits = jnp.dot(gr, mw_ref[...],
                         preferred_element_type=jnp.float32) + mb_ref[...]
        m = jnp.max(logits, axis=1, keepdims=True)
        lse = jnp.log(jnp.sum(jnp.exp(logits - m), axis=1, keepdims=True)) + m
        o_ref[...] = logits - lse


def _final(aggp, x1, degp, graph_ids, W2, b2, mlp_W, mlp_b):
    gid_r = graph_ids.reshape(N // BM, 1, BM)
    return pl.pallas_call(
        _final_body,
        grid=(N // BM,),
        in_specs=[pl.BlockSpec((NC, BM, D), lambda i: (0, i, 0)),
                  pl.BlockSpec((BM, D), lambda i: (i, 0)),
                  pl.BlockSpec((NC, BM, DW), lambda i: (0, i, 0)),
                  pl.BlockSpec((1, 1, BM), lambda i: (i, 0, 0)),
                  pl.BlockSpec((D, D), lambda i: (0, 0)),
                  pl.BlockSpec((1, D), lambda i: (0, 0)),
                  pl.BlockSpec((D, NCLS), lambda i: (0, 0)),
                  pl.BlockSpec((1, NCLS), lambda i: (0, 0))],
        out_specs=pl.BlockSpec((NG, NCLS), lambda i: (0, 0)),
        out_shape=jax.ShapeDtypeStruct((NG, NCLS), jnp.float32),
        scratch_shapes=[pltpu.VMEM((NG, D), jnp.float32),
                        pltpu.VMEM((NG, D), jnp.float32)],
    )(aggp, x1, degp, gid_r, W2, b2.reshape(1, D), mlp_W,
      mlp_b.reshape(1, NCLS))


def kernel(node_feat, W1, b1, W2, b2, mlp_W, mlp_b, edge_index, graph_ids):
    src = edge_index[0]
    dst = edge_index[1]
    znd = jnp.zeros((NPAD, D), jnp.float32)
    zdw = jnp.zeros((NPAD, DW), jnp.float32)
    ones = jnp.ones((K, DW), jnp.float32)

    support1 = _matmul(node_feat, W1)
    degp = _sc_deg(dst, ones, zdw)
    aggp1 = _sc_agg(support1, src, dst, znd)
    x1 = _combine_relu(aggp1, support1, degp, b1)
    aggp2 = _sc_agg(x1, src, dst, znd)
    return _final(aggp2, x1, degp, graph_ids, W2, b2, mlp_W, mlp_b)
